# baseline (speedup 1.0000x reference)
import functools

import jax
import jax.numpy as jnp
from jax import lax
from jax.experimental import pallas as pl
from jax.experimental.pallas import tpu as pltpu
from jax.experimental.pallas import tpu_sc as plsc

L = 16


def _sampler_body(nop, rows_per_w, nchunks, nc,
                  flat_hbm, weight_hbm, sv_hbm, out_hbm,
                  wv, svv, idxv, rowsv, sem):
    wid = lax.axis_index("s") * nc + lax.axis_index("c")
    base = wid * rows_per_w

    pltpu.sync_copy(weight_hbm.at[pl.ds(base, rows_per_w)], wv)
    pltpu.sync_copy(sv_hbm.at[pl.ds(base, rows_per_w)], svv)

    lane = jnp.arange(L, dtype=jnp.int32)
    zero_i = jnp.zeros((L,), jnp.int32)

    row_vals = []
    for r in range(rows_per_w):
        b = base + r
        row_vals.append(zero_i)

    vals = row_vals[0]
    for r in range(1, rows_per_w):
        vals = jnp.where(lane == r, row_vals[r], vals)
    plsc.store_scatter(idxv, [jnp.where(lane < rows_per_w, lane, 0)],
                       vals, mask=lane < rows_per_w)

    pltpu.async_copy(flat_hbm.at[idxv], rowsv, sem).wait()
    pltpu.sync_copy(rowsv, out_hbm.at[pl.ds(base, rows_per_w)])


def kernel(inputs, weight):
    B, nop, D = inputs.shape
    sv = jax.random.uniform(jax.random.key(42), (B, 1),
                            dtype=weight.dtype)
    sv = jnp.broadcast_to(sv, (B, L))
    flat = inputs.reshape(B * nop, D)[:8]

    info = plsc.get_sparse_core_info()
    nc, ns = info.num_cores, info.num_subcores
    nw = nc * ns
    rows_per_w = B // nw
    nchunks = nop // L

    mesh = plsc.VectorSubcoreMesh(core_axis_name="c", subcore_axis_name="s")
    k = functools.partial(
        pl.kernel,
        mesh=mesh,
        compiler_params=pltpu.CompilerParams(needs_layout_passes=False),
        out_type=jax.ShapeDtypeStruct((B, D), inputs.dtype),
        scratch_types=[
            pltpu.VMEM((rows_per_w, nop), jnp.float32),
            pltpu.VMEM((rows_per_w, L), jnp.float32),
            pltpu.VMEM((rows_per_w,), jnp.int32),
            pltpu.VMEM((rows_per_w, D), jnp.float32),
            pltpu.SemaphoreType.DMA,
        ],
    )(functools.partial(_sampler_body, nop, rows_per_w, nchunks, nc))
    return k(flat, weight, sv)


# speedup vs baseline: 4.9591x; 4.9591x over previous
import functools

import jax
import jax.numpy as jnp
from jax.experimental import pallas as pl
from jax.experimental.pallas import tpu as pltpu


def _body(B, nop, D, inputs_hbm, weight_ref, sv_ref, out_ref,
          ind_vmem, ind_smem, sem_i, sem_rows):
    w = weight_ref[...]
    x = w
    k = 1
    while k < nop:
        shifted = jnp.concatenate(
            [jnp.zeros((B, k), jnp.float32), x[:, :nop - k]], axis=1)
        x = x + shifted
        k *= 2
    mask = (x < sv_ref[...]).astype(jnp.int32)
    cnt = jnp.sum(mask, axis=1)
    ind = jnp.where(cnt == nop, 0, cnt)
    ind_vmem[...] = ind
    pltpu.async_copy(ind_vmem, ind_smem, sem_i).wait()

    copies = []
    for b in range(B):
        ib = ind_smem[b]
        copies.append(
            pltpu.async_copy(inputs_hbm.at[b, ib], out_ref.at[b], sem_rows))
    for c in copies:
        c.wait()


def kernel(inputs, weight):
    B, nop, D = inputs.shape
    sv = jax.random.uniform(jax.random.key(42), (B, 1), dtype=weight.dtype)

    return pl.pallas_call(
        functools.partial(_body, B, nop, D),
        in_specs=[
            pl.BlockSpec(memory_space=pltpu.HBM),
            pl.BlockSpec(memory_space=pltpu.VMEM),
            pl.BlockSpec(memory_space=pltpu.VMEM),
        ],
        out_specs=pl.BlockSpec(memory_space=pltpu.VMEM),
        out_shape=jax.ShapeDtypeStruct((B, D), inputs.dtype),
        scratch_shapes=[
            pltpu.VMEM((B,), jnp.int32),
            pltpu.SMEM((B,), jnp.int32),
            pltpu.SemaphoreType.DMA,
            pltpu.SemaphoreType.DMA,
        ],
    )(inputs, weight, sv)


# speedup vs baseline: 5.9105x; 1.1918x over previous
import functools

import jax
import jax.numpy as jnp
import numpy as np
from jax.experimental import pallas as pl
from jax.experimental.pallas import tpu as pltpu

_SV_CACHE = {}


def _threshold_constant(B, dtype):
    key = (B, jnp.dtype(dtype).name)
    if key not in _SV_CACHE:
        with jax.ensure_compile_time_eval():
            _SV_CACHE[key] = np.asarray(
                jax.random.uniform(jax.random.key(42), (B, 1), dtype=dtype))
    return _SV_CACHE[key]


def _body(B, nop, D, inputs_hbm, weight_ref, sv_ref, out_ref,
          ind_vmem, ind_smem, sem_i, sem_rows):
    w = weight_ref[...]
    x = w
    k = 1
    while k < nop:
        shifted = jnp.concatenate(
            [jnp.zeros((B, k), jnp.float32), x[:, :nop - k]], axis=1)
        x = x + shifted
        k *= 2
    mask = (x < sv_ref[...]).astype(jnp.int32)
    cnt = jnp.sum(mask, axis=1)
    ind = jnp.where(cnt == nop, 0, cnt)
    ind_vmem[...] = ind
    pltpu.async_copy(ind_vmem, ind_smem, sem_i).wait()

    copies = []
    for b in range(B):
        ib = ind_smem[b]
        copies.append(
            pltpu.async_copy(inputs_hbm.at[b, ib], out_ref.at[b], sem_rows))
    for c in copies:
        c.wait()


def kernel(inputs, weight):
    B, nop, D = inputs.shape
    sv = jnp.asarray(_threshold_constant(B, weight.dtype))

    return pl.pallas_call(
        functools.partial(_body, B, nop, D),
        in_specs=[
            pl.BlockSpec(memory_space=pltpu.HBM),
            pl.BlockSpec(memory_space=pltpu.VMEM),
            pl.BlockSpec(memory_space=pltpu.VMEM),
        ],
        out_specs=pl.BlockSpec(memory_space=pltpu.VMEM),
        out_shape=jax.ShapeDtypeStruct((B, D), inputs.dtype),
        scratch_shapes=[
            pltpu.VMEM((B,), jnp.int32),
            pltpu.SMEM((B,), jnp.int32),
            pltpu.SemaphoreType.DMA,
            pltpu.SemaphoreType.DMA,
        ],
    )(inputs, weight, sv)
